# baseline (speedup 1.0000x reference)
import functools
import math

import jax
import jax.numpy as jnp
from jax.experimental import pallas as pl
from jax.experimental.pallas import tpu as pltpu
from jax.experimental.pallas import tpu_sc as plsc

EMB_DIM = 32
SCALE = math.sqrt(float(EMB_DIM))
LANES = 16
N_L = 200
N_B = 4096
W = 512


def _build_gather():
    mesh = plsc.VectorSubcoreMesh(core_axis_name="c", subcore_axis_name="s")
    cp = pltpu.CompilerParams(
        needs_layout_passes=False, use_tc_tiling_on_sc=False
    )
    n = N_L * N_B

    @functools.partial(
        pl.kernel,
        out_type=jax.ShapeDtypeStruct((n, EMB_DIM), jnp.float32),
        mesh=mesh,
        compiler_params=cp,
        scratch_types=[pltpu.VMEM((W,), jnp.int32)],
    )
    def gather_kernel(table_hbm, idx_hbm, out_hbm, i4_scr):
        iota16 = jax.lax.iota(jnp.int32, LANES)

        def body(i_vmem, o_vmem):
            for k in range(W // LANES):
                q = iota16 + LANES * k
                src = 128 * (q % 4) + q // 4
                t16 = plsc.load_gather(
                    i_vmem, [jnp.zeros((LANES,), jnp.int32), src]
                )
                i4_scr[pl.ds(LANES * k, LANES)] = t16 * 4

            pltpu.sync_copy(table_hbm.at[i4_scr], o_vmem)

        pltpu.emit_pipeline(
            body,
            grid=(n // W,),
            in_specs=[pl.BlockSpec((1, W), lambda i: (0, i))],
            out_specs=[pl.BlockSpec((W, EMB_DIM), lambda i: (i, 0))],
            core_axis_name=("c", "s"),
            dimension_semantics=(pltpu.PARALLEL,),
        )(idx_hbm, out_hbm)

    return gather_kernel


def _detile_body(t_ref, o_ref):
    y = t_ref[...].T
    o_ref[...] = jnp.concatenate(
        [y, jnp.zeros((512, 96), jnp.float32)], axis=1
    )


def _build_detile():
    return pl.pallas_call(
        _detile_body,
        grid=(1954,),
        in_specs=[pl.BlockSpec((EMB_DIM, 512), lambda i: (0, i))],
        out_specs=pl.BlockSpec((512, 128), lambda i: (i, 0)),
        out_shape=jax.ShapeDtypeStruct((1000000, 128), jnp.float32),
        compiler_params=pltpu.CompilerParams(
            dimension_semantics=("arbitrary",)
        ),
    )


def _finish_body(g_ref, tok_ref, o_ref):
    for w in range(4):
        x = g_ref[0, w]
        y = x.T
        tok = tok_ref[0, w]
        scale = jnp.where(tok == 1, 0.0, SCALE).astype(jnp.float32)
        for jj in range(4):
            blk = y[32 * jj : 32 * (jj + 1), :] * scale[jj][None, :]
            o_ref[0, :, 4 * w + jj, :, :] = blk.reshape(4, 8, 128)


def _build_finish():
    return pl.pallas_call(
        _finish_body,
        grid=(N_L, 2),
        in_specs=[
            pl.BlockSpec((1, 4, 128, 128), lambda l, t: (l, t, 0, 0)),
            pl.BlockSpec((1, 4, 4, 128), lambda l, t: (l, t, 0, 0)),
        ],
        out_specs=pl.BlockSpec(
            (1, 4, 16, 8, 128), lambda l, t: (l, 0, t, 0, 0)
        ),
        out_shape=jax.ShapeDtypeStruct((N_L, 4, 32, 8, 128), jnp.float32),
        compiler_params=pltpu.CompilerParams(
            dimension_semantics=("parallel", "parallel")
        ),
    )


def kernel(tokens, table):
    n = N_L * N_B
    tokT = tokens.T.astype(jnp.int32)
    idx = tokT.reshape(1, n)
    tok4 = tokT.reshape(N_L, 8, 4, 128)
    table_pad = _build_detile()(table.T).reshape(4 * 1000000, EMB_DIM)
    g = _build_gather()(table_pad, idx)
    g4 = g.reshape(N_L, 8, 128, 128)
    out5 = _build_finish()(g4, tok4)
    return out5.transpose(2, 4, 0, 1, 3).reshape(N_B, N_L, EMB_DIM)


# speedup vs baseline: 1.8002x; 1.8002x over previous
import functools
import math

import jax
import jax.numpy as jnp
from jax.experimental import pallas as pl
from jax.experimental.pallas import tpu as pltpu
from jax.experimental.pallas import tpu_sc as plsc

EMB_DIM = 32
SCALE = math.sqrt(float(EMB_DIM))
LANES = 16
N_L = 200
N_B = 4096
W = 512


def _build_gather():
    mesh = plsc.VectorSubcoreMesh(core_axis_name="c", subcore_axis_name="s")
    cp = pltpu.CompilerParams(
        needs_layout_passes=False, use_tc_tiling_on_sc=False
    )
    n = N_L * N_B

    @functools.partial(
        pl.kernel,
        out_type=jax.ShapeDtypeStruct((n, EMB_DIM), jnp.float32),
        mesh=mesh,
        compiler_params=cp,
        scratch_types=[pltpu.VMEM((W,), jnp.int32)],
    )
    def gather_kernel(table_hbm, idx_hbm, out_hbm, i4_scr):
        iota16 = jax.lax.iota(jnp.int32, LANES)

        def body(i_vmem, o_vmem):
            for k in range(W // LANES):
                q = iota16 + LANES * k
                src = 128 * (q % 4) + q // 4
                t16 = plsc.load_gather(
                    i_vmem, [jnp.zeros((LANES,), jnp.int32), src]
                )
                i4_scr[pl.ds(LANES * k, LANES)] = t16 * 4

            pltpu.sync_copy(table_hbm.at[i4_scr], o_vmem)

        pltpu.emit_pipeline(
            body,
            grid=(n // W,),
            in_specs=[pl.BlockSpec((1, W), lambda i: (0, i))],
            out_specs=[pl.BlockSpec((W, EMB_DIM), lambda i: (i, 0))],
            core_axis_name=("c", "s"),
            dimension_semantics=(pltpu.PARALLEL,),
        )(idx_hbm, out_hbm)

    return gather_kernel


def _finish_body(g_ref, tok_ref, o_ref):
    for w in range(4):
        x = g_ref[0, w]
        y = x.T
        tok = tok_ref[0, w]
        scale = jnp.where(tok == 1, 0.0, SCALE).astype(jnp.float32)
        for jj in range(4):
            blk = y[32 * jj : 32 * (jj + 1), :] * scale[jj][None, :]
            o_ref[0, :, 4 * w + jj, :, :] = blk.reshape(4, 8, 128)


def _build_finish():
    return pl.pallas_call(
        _finish_body,
        grid=(N_L, 2),
        in_specs=[
            pl.BlockSpec((1, 4, 128, 128), lambda l, t: (l, t, 0, 0)),
            pl.BlockSpec((1, 4, 4, 128), lambda l, t: (l, t, 0, 0)),
        ],
        out_specs=pl.BlockSpec(
            (1, 4, 16, 8, 128), lambda l, t: (l, 0, t, 0, 0)
        ),
        out_shape=jax.ShapeDtypeStruct((N_L, 4, 32, 8, 128), jnp.float32),
        compiler_params=pltpu.CompilerParams(
            dimension_semantics=("parallel", "parallel")
        ),
    )


def kernel(tokens, table):
    n = N_L * N_B
    tokT = tokens.T.astype(jnp.int32)
    idx = tokT.reshape(1, n)
    tok4 = tokT.reshape(N_L, 8, 4, 128)
    table_pad = jnp.pad(table, ((0, 0), (0, 96))).reshape(4 * 1000000, EMB_DIM)
    g = _build_gather()(table_pad, idx)
    g4 = g.reshape(N_L, 8, 128, 128)
    out5 = _build_finish()(g4, tok4)
    return out5.transpose(2, 4, 0, 1, 3).reshape(N_B, N_L, EMB_DIM)


# speedup vs baseline: 2.0458x; 1.1364x over previous
import functools
import math

import jax
import jax.numpy as jnp
from jax.experimental import pallas as pl
from jax.experimental.pallas import tpu as pltpu
from jax.experimental.pallas import tpu_sc as plsc

EMB_DIM = 32
SCALE = math.sqrt(float(EMB_DIM))
LANES = 16
N_L = 200
N_B = 4096
W = 512


def _build_gather():
    mesh = plsc.VectorSubcoreMesh(core_axis_name="c", subcore_axis_name="s")
    cp = pltpu.CompilerParams(
        needs_layout_passes=False, use_tc_tiling_on_sc=False
    )
    n = N_L * N_B

    @functools.partial(
        pl.kernel,
        out_type=jax.ShapeDtypeStruct((n, EMB_DIM), jnp.float32),
        mesh=mesh,
        compiler_params=cp,
        scratch_types=[pltpu.VMEM((W,), jnp.int32)],
    )
    def gather_kernel(table_hbm, idx_hbm, out_hbm, i4_scr):
        iota16 = jax.lax.iota(jnp.int32, LANES)

        def body(i_vmem, o_vmem):
            for k in range(W // LANES):
                q = iota16 + LANES * k
                src = 128 * (q % 4) + q // 4
                t16 = plsc.load_gather(
                    i_vmem, [jnp.zeros((LANES,), jnp.int32), src]
                )
                i4_scr[pl.ds(LANES * k, LANES)] = t16 * 4

            pltpu.sync_copy(table_hbm.at[i4_scr], o_vmem)

        pltpu.emit_pipeline(
            body,
            grid=(n // W,),
            in_specs=[pl.BlockSpec((1, W), lambda i: (0, i))],
            out_specs=[pl.BlockSpec((W, EMB_DIM), lambda i: (i, 0))],
            core_axis_name=("c", "s"),
            dimension_semantics=(pltpu.PARALLEL,),
        )(idx_hbm, out_hbm)

    return gather_kernel


def _finish_body(g_ref, tok_ref, o_ref):
    for w in range(8):
        x = g_ref[0, w]
        y = x.T
        tok = tok_ref[0, w]
        scale = jnp.where(tok == 1, 0.0, SCALE).astype(jnp.float32)
        for jj in range(4):
            blk = y[32 * jj : 32 * (jj + 1), :] * scale[jj][None, :]
            o_ref[0, :, 4 * w + jj, :, :] = blk.reshape(4, 8, 128)


def _build_finish():
    return pl.pallas_call(
        _finish_body,
        grid=(N_L,),
        in_specs=[
            pl.BlockSpec((1, 8, 128, 128), lambda l: (l, 0, 0, 0)),
            pl.BlockSpec((1, 8, 4, 128), lambda l: (l, 0, 0, 0)),
        ],
        out_specs=pl.BlockSpec(
            (1, 4, 32, 8, 128), lambda l: (l, 0, 0, 0, 0)
        ),
        out_shape=jax.ShapeDtypeStruct((N_L, 4, 32, 8, 128), jnp.float32),
        compiler_params=pltpu.CompilerParams(
            dimension_semantics=("parallel",)
        ),
    )


def kernel(tokens, table):
    n = N_L * N_B
    tokT = tokens.T.astype(jnp.int32)
    idx = tokT.reshape(1, n)
    tok4 = tokT.reshape(N_L, 8, 4, 128)
    table_pad = jnp.pad(table, ((0, 0), (0, 96))).reshape(4 * 1000000, EMB_DIM)
    g = _build_gather()(table_pad, idx)
    g4 = g.reshape(N_L, 8, 128, 128)
    out5 = _build_finish()(g4, tok4)
    return out5.transpose(2, 4, 0, 1, 3).reshape(N_B, N_L, EMB_DIM)
